# Initial kernel scaffold; baseline (speedup 1.0000x reference)
import jax
import jax.numpy as jnp
from jax.experimental import pallas as pl


def kernel(inputs, table):
    raise NotImplementedError("write your pallas kernel here")


# speedup vs baseline: 7.1585x; 7.1585x over previous
import functools

import jax
import jax.numpy as jnp
from jax import lax
from jax.experimental import pallas as pl
from jax.experimental.pallas import tpu as pltpu
from jax.experimental.pallas import tpu_sc as plsc

B, F, L, E = 4096, 26, 50, 32
S = B * F
NC, NS = 2, 16
NW = NC * NS
SEG_PER_W = S // NW
CHUNK_SEGS = 2
CHUNK_IDX = CHUNK_SEGS * L
CHUNKS_PER_W = SEG_PER_W // CHUNK_SEGS

_GATHER_DNUMS = lax.GatherDimensionNumbers(
    offset_dims=(), collapsed_slice_dims=(0,), start_index_map=(0,))


def _bcast_lane(v, lane):
    idx = jnp.full((16, 1), lane, dtype=jnp.int32)
    return lax.gather(v, idx, _GATHER_DNUMS, (1,),
                      mode=lax.GatherScatterMode.PROMISE_IN_BOUNDS)


def _seg_accumulate(idx_v, rows_v, out_v, s):
    base = s * L
    offs = (base, base + 16, base + 32, base + 34)
    masks = []
    for o in offs:
        iv = idx_v[pl.ds(o, 16)]
        masks.append(jnp.where(iv != 0, 1.0, 0.0).astype(jnp.float32))
    acc0 = jnp.zeros((16,), jnp.float32)
    acc1 = jnp.zeros((16,), jnp.float32)
    for j in range(L):
        if j < 48:
            c, lane = j // 16, j % 16
        else:
            c, lane = 3, j - 34
        mj = _bcast_lane(masks[c], lane)
        r0 = rows_v[base + j, pl.ds(0, 16)]
        r1 = rows_v[base + j, pl.ds(16, 16)]
        acc0 = acc0 + r0 * mj
        acc1 = acc1 + r1 * mj
    out_v[s, pl.ds(0, 16)] = acc0
    out_v[s, pl.ds(16, 16)] = acc1


@functools.partial(
    pl.kernel,
    out_type=jax.ShapeDtypeStruct((NW, CHUNKS_PER_W, CHUNK_SEGS, E),
                                  jnp.float32),
    mesh=plsc.VectorSubcoreMesh(core_axis_name="c", subcore_axis_name="s"),
    scratch_types=[
        pltpu.VMEM((CHUNK_IDX,), jnp.int32),
        pltpu.VMEM((CHUNK_IDX, E), jnp.float32),
        pltpu.VMEM((CHUNK_SEGS, E), jnp.float32),
        pltpu.SemaphoreType.DMA,
    ],
    compiler_params=pltpu.CompilerParams(use_tc_tiling_on_sc=False),
)
def _flsum_kernel(idx_hbm, table_hbm, out_hbm, idx_v, rows_v, out_v, sem):
    wid = lax.axis_index("s") * NC + lax.axis_index("c")

    @pl.loop(0, CHUNKS_PER_W)
    def _chunk(c):
        pltpu.sync_copy(idx_hbm.at[wid, c], idx_v)
        pltpu.async_copy(table_hbm.at[idx_v], rows_v, sem).wait()
        for s in range(CHUNK_SEGS):
            _seg_accumulate(idx_v, rows_v, out_v, s)
        pltpu.sync_copy(out_v, out_hbm.at[wid, c])


def kernel(inputs, table):
    idx = inputs.reshape(NW, CHUNKS_PER_W, CHUNK_IDX)
    out = _flsum_kernel(idx, table)
    return out.reshape(B, F, E)
